# baseline (speedup 1.0000x reference)
import jax
import jax.numpy as jnp
from jax import lax
from jax.experimental import pallas as pl
from jax.experimental.pallas import tpu as pltpu
from jax.experimental.pallas import tpu_sc as plsc

B, C, H, W = 32, 768, 32, 32
P = H * W
K = 8
LANES = 16
CHUNK = 128
GROUPS = CHUNK // LANES
NCHUNKS = P // CHUNK
CBATCH = C // K
NC, NS = 2, 16
CBLK = 128

_SORT8 = [[(0, 1), (2, 3), (4, 5), (6, 7)],
          [(0, 2), (1, 3), (4, 6), (5, 7)],
          [(1, 2), (5, 6)],
          [(0, 4), (1, 5), (2, 6), (3, 7)],
          [(2, 4), (3, 5)],
          [(1, 2), (3, 4), (5, 6)]]
_BITONIC8 = [[(0, 4), (1, 5), (2, 6), (3, 7)],
             [(0, 2), (1, 3), (4, 6), (5, 7)],
             [(0, 1), (2, 3), (4, 5), (6, 7)]]


def _apply_net(vals, net):
    for layer in net:
        for a, b in layer:
            hi = jnp.maximum(vals[a], vals[b])
            lo = jnp.minimum(vals[a], vals[b])
            vals[a], vals[b] = hi, lo
    return vals


def _sc_thresholds(x_hbm, thr_hbm, buf, thr_buf, sem, out_sem):
    wid = lax.axis_index("s") * NC + lax.axis_index("c")

    @pl.loop(0, NCHUNKS)
    def _chunk(q):
        pltpu.async_copy(x_hbm.at[wid, :, pl.ds(q * CHUNK, CHUNK)],
                         buf, sem).wait()

        neg = jnp.full((LANES,), -jnp.inf, jnp.float32)

        for gbase in range(0, GROUPS, 2):
            def batch_body(c8, ms, gbase=gbase):
                ms = list(ms)
                base = c8 * K
                for gg in range(2):
                    g = gbase + gg
                    t = [buf[base + k, g * LANES:(g + 1) * LANES]
                         for k in range(K)]
                    t = _apply_net(t, _SORT8)
                    m = ms[gg * K:(gg + 1) * K]
                    u = [jnp.maximum(m[i], t[K - 1 - i]) for i in range(K)]
                    u = _apply_net(u, _BITONIC8)
                    ms[gg * K:(gg + 1) * K] = u
                return tuple(ms)

            ms = plsc.parallel_loop(
                0, CBATCH,
                carry=tuple(neg for _ in range(2 * K)))(batch_body)
            for gg in range(2):
                g = gbase + gg
                thr_buf[pl.ds(q * CHUNK + g * LANES, LANES)] = (
                    ms[gg * K + K - 1])

    pltpu.async_copy(thr_buf, thr_hbm.at[wid], out_sem).wait()


def _tc_mask(x_ref, thr_ref, out_ref):
    xb = x_ref[0]
    t = thr_ref[0]
    out_ref[0] = jnp.where(xb >= t, xb, jnp.float32(0.0))


@jax.jit
def kernel(x):
    xr = x.reshape(B, C, P)
    thr = pl.kernel(
        _sc_thresholds,
        out_type=jax.ShapeDtypeStruct((B, P), jnp.float32),
        mesh=plsc.VectorSubcoreMesh(core_axis_name="c", subcore_axis_name="s"),
        scratch_types=[
            pltpu.VMEM((C, CHUNK), jnp.float32),
            pltpu.VMEM((P,), jnp.float32),
            pltpu.SemaphoreType.DMA,
            pltpu.SemaphoreType.DMA,
        ],
        compiler_params=pltpu.CompilerParams(use_tc_tiling_on_sc=True),
    )(xr)
    out = pl.pallas_call(
        _tc_mask,
        grid=(B, C // CBLK),
        in_specs=[
            pl.BlockSpec((1, CBLK, H, W), lambda b, cb: (b, cb, 0, 0)),
            pl.BlockSpec((1, 1, H, W), lambda b, cb: (b, 0, 0, 0)),
        ],
        out_specs=pl.BlockSpec((1, CBLK, H, W), lambda b, cb: (b, cb, 0, 0)),
        out_shape=jax.ShapeDtypeStruct((B, C, H, W), jnp.float32),
    )(x, thr.reshape(B, 1, H, W))
    return out


# speedup vs baseline: 1.4349x; 1.4349x over previous
import jax
import jax.numpy as jnp
from jax import lax
from jax.experimental import pallas as pl
from jax.experimental.pallas import tpu as pltpu
from jax.experimental.pallas import tpu_sc as plsc

B, C, H, W = 32, 768, 32, 32
P = H * W
K = 8
LANES = 16
CHUNK = 128
GROUPS = CHUNK // LANES
NCHUNKS = P // CHUNK
CBATCH = C // K
NC, NS = 2, 16
CBLK = 384

_SORT8 = [[(0, 1), (2, 3), (4, 5), (6, 7)],
          [(0, 2), (1, 3), (4, 6), (5, 7)],
          [(1, 2), (5, 6)],
          [(0, 4), (1, 5), (2, 6), (3, 7)],
          [(2, 4), (3, 5)],
          [(1, 2), (3, 4), (5, 6)]]
_BITONIC8 = [[(0, 4), (1, 5), (2, 6), (3, 7)],
             [(0, 2), (1, 3), (4, 6), (5, 7)],
             [(0, 1), (2, 3), (4, 5), (6, 7)]]


def _apply_net(vals, net):
    for layer in net:
        for a, b in layer:
            hi = jnp.maximum(vals[a], vals[b])
            lo = jnp.minimum(vals[a], vals[b])
            vals[a], vals[b] = hi, lo
    return vals


def _sc_thresholds(x_hbm, thr_hbm, buf, thr_buf, sem, out_sem):
    wid = lax.axis_index("s") * NC + lax.axis_index("c")

    @pl.loop(0, NCHUNKS)
    def _chunk(q):
        pltpu.async_copy(x_hbm.at[wid, :, pl.ds(q * CHUNK, CHUNK)],
                         buf, sem).wait()

        neg = jnp.full((LANES,), -jnp.inf, jnp.float32)

        for gbase in range(0, GROUPS, 2):
            def batch_body(c8, ms, gbase=gbase):
                ms = list(ms)
                base = c8 * K
                for gg in range(2):
                    g = gbase + gg
                    t = [buf[base + k, g * LANES:(g + 1) * LANES]
                         for k in range(K)]
                    t = _apply_net(t, _SORT8)
                    m = ms[gg * K:(gg + 1) * K]
                    u = [jnp.maximum(m[i], t[K - 1 - i]) for i in range(K)]
                    u = _apply_net(u, _BITONIC8)
                    ms[gg * K:(gg + 1) * K] = u
                return tuple(ms)

            ms = plsc.parallel_loop(
                0, CBATCH,
                carry=tuple(neg for _ in range(2 * K)))(batch_body)
            for gg in range(2):
                g = gbase + gg
                thr_buf[pl.ds(q * CHUNK + g * LANES, LANES)] = (
                    ms[gg * K + K - 1])

    pltpu.async_copy(thr_buf, thr_hbm.at[wid], out_sem).wait()


def _tc_mask(x_ref, thr_ref, out_ref):
    xb = x_ref[...]
    t = thr_ref[0]
    out_ref[...] = jnp.where(xb >= t, xb, jnp.float32(0.0))


@jax.jit
def kernel(x):
    xr = x.reshape(B, C, P)
    thr = pl.kernel(
        _sc_thresholds,
        out_type=jax.ShapeDtypeStruct((B, P), jnp.float32),
        mesh=plsc.VectorSubcoreMesh(core_axis_name="c", subcore_axis_name="s"),
        scratch_types=[
            pltpu.VMEM((C, CHUNK), jnp.float32),
            pltpu.VMEM((P,), jnp.float32),
            pltpu.SemaphoreType.DMA,
            pltpu.SemaphoreType.DMA,
        ],
        compiler_params=pltpu.CompilerParams(use_tc_tiling_on_sc=True),
    )(xr)
    x2 = xr.reshape(B * C, P)
    nblk = B * C // CBLK
    blk_per_b = C // CBLK
    out = pl.pallas_call(
        _tc_mask,
        grid=(nblk,),
        in_specs=[
            pl.BlockSpec((CBLK, P), lambda i: (i, 0)),
            pl.BlockSpec((1, 1, P), lambda i: (i // blk_per_b, 0, 0)),
        ],
        out_specs=pl.BlockSpec((CBLK, P), lambda i: (i, 0)),
        out_shape=jax.ShapeDtypeStruct((B * C, P), jnp.float32),
        compiler_params=pltpu.CompilerParams(
            dimension_semantics=("arbitrary",)),
    )(x2, thr.reshape(B, 1, P))
    return out.reshape(B, C, H, W)


# speedup vs baseline: 2.0944x; 1.4596x over previous
import jax
import jax.numpy as jnp
from jax import lax
from jax.experimental import pallas as pl
from jax.experimental.pallas import tpu as pltpu
from jax.experimental.pallas import tpu_sc as plsc

B, C, H, W = 32, 768, 32, 32
P = H * W
K = 8
LANES = 16
CHUNK = 128
GROUPS = CHUNK // LANES
NCHUNKS = P // CHUNK
CBATCH = C // K
NC, NS = 2, 16
CBLK = 128

_SORT8 = [[(0, 1), (2, 3), (4, 5), (6, 7)],
          [(0, 2), (1, 3), (4, 6), (5, 7)],
          [(1, 2), (5, 6)],
          [(0, 4), (1, 5), (2, 6), (3, 7)],
          [(2, 4), (3, 5)],
          [(1, 2), (3, 4), (5, 6)]]
_BITONIC8 = [[(0, 4), (1, 5), (2, 6), (3, 7)],
             [(0, 2), (1, 3), (4, 6), (5, 7)],
             [(0, 1), (2, 3), (4, 5), (6, 7)]]


def _apply_net(vals, net):
    for layer in net:
        for a, b in layer:
            hi = jnp.maximum(vals[a], vals[b])
            lo = jnp.minimum(vals[a], vals[b])
            vals[a], vals[b] = hi, lo
    return vals


HALFB = B // 2
HALFP = P // 2
NCHUNKS_H = HALFP // CHUNK


def _sc_thresholds(x_hbm, thr_hbm, buf, thr_buf, sem, out_sem):
    wid = lax.axis_index("s") * NC + lax.axis_index("c")
    bat = wid // 2
    poff = (wid % 2) * HALFP

    @pl.loop(0, NCHUNKS_H)
    def _chunk(q):
        pltpu.async_copy(x_hbm.at[bat, :, pl.ds(poff + q * CHUNK, CHUNK)],
                         buf, sem).wait()

        neg = jnp.full((LANES,), -jnp.inf, jnp.float32)

        for gbase in range(0, GROUPS, 2):
            def batch_body(c8, ms, gbase=gbase):
                ms = list(ms)
                base = c8 * K
                for gg in range(2):
                    g = gbase + gg
                    t = [buf[base + k, g * LANES:(g + 1) * LANES]
                         for k in range(K)]
                    t = _apply_net(t, _SORT8)
                    m = ms[gg * K:(gg + 1) * K]
                    u = [jnp.maximum(m[i], t[K - 1 - i]) for i in range(K)]
                    u = _apply_net(u, _BITONIC8)
                    ms[gg * K:(gg + 1) * K] = u
                return tuple(ms)

            ms = plsc.parallel_loop(
                0, CBATCH,
                carry=tuple(neg for _ in range(2 * K)))(batch_body)
            for gg in range(2):
                g = gbase + gg
                thr_buf[pl.ds(q * CHUNK + g * LANES, LANES)] = (
                    ms[gg * K + K - 1])

    pltpu.async_copy(thr_buf, thr_hbm.at[bat, pl.ds(poff, HALFP)],
                     out_sem).wait()


def _tc_mask(x_ref, thr_ref, out_ref):
    xb = x_ref[0]
    t = thr_ref[0]
    out_ref[0] = jnp.where(xb >= t, xb, jnp.float32(0.0))


def _sc_call(xh):
    return pl.kernel(
        _sc_thresholds,
        out_type=jax.ShapeDtypeStruct((HALFB, P), jnp.float32),
        mesh=plsc.VectorSubcoreMesh(core_axis_name="c", subcore_axis_name="s"),
        scratch_types=[
            pltpu.VMEM((C, CHUNK), jnp.float32),
            pltpu.VMEM((HALFP,), jnp.float32),
            pltpu.SemaphoreType.DMA,
            pltpu.SemaphoreType.DMA,
        ],
        compiler_params=pltpu.CompilerParams(use_tc_tiling_on_sc=True),
    )(xh)


def _mask_call(xh, thrh):
    return pl.pallas_call(
        _tc_mask,
        grid=(HALFB, C // CBLK),
        in_specs=[
            pl.BlockSpec((1, CBLK, P), lambda b, cb: (b, cb, 0)),
            pl.BlockSpec((1, 1, P), lambda b, cb: (b, 0, 0)),
        ],
        out_specs=pl.BlockSpec((1, CBLK, P), lambda b, cb: (b, cb, 0)),
        out_shape=jax.ShapeDtypeStruct((HALFB, C, P), jnp.float32),
    )(xh, thrh.reshape(HALFB, 1, P))


@jax.jit
def kernel(x):
    xr = x.reshape(B, C, P)
    xa, xb = xr[:HALFB], xr[HALFB:]
    thra = _sc_call(xa)
    thrb = _sc_call(xb)
    outa = _mask_call(xa, thra)
    outb = _mask_call(xb, thrb)
    return jnp.concatenate([outa, outb], axis=0).reshape(B, C, H, W)


# speedup vs baseline: 2.3382x; 1.1164x over previous
import jax
import jax.numpy as jnp
from jax import lax
from jax.experimental import pallas as pl
from jax.experimental.pallas import tpu as pltpu
from jax.experimental.pallas import tpu_sc as plsc

B, C, H, W = 32, 768, 32, 32
P = H * W
K = 8
LANES = 16
CHUNK = 128
GROUPS = CHUNK // LANES
NCHUNKS = P // CHUNK
CBATCH = C // K
NC, NS = 2, 16
CBLK = 128

_SORT8 = [[(0, 1), (2, 3), (4, 5), (6, 7)],
          [(0, 2), (1, 3), (4, 6), (5, 7)],
          [(1, 2), (5, 6)],
          [(0, 4), (1, 5), (2, 6), (3, 7)],
          [(2, 4), (3, 5)],
          [(1, 2), (3, 4), (5, 6)]]
_BITONIC8 = [[(0, 4), (1, 5), (2, 6), (3, 7)],
             [(0, 2), (1, 3), (4, 6), (5, 7)],
             [(0, 1), (2, 3), (4, 5), (6, 7)]]


def _apply_net(vals, net):
    for layer in net:
        for a, b in layer:
            hi = jnp.maximum(vals[a], vals[b])
            lo = jnp.minimum(vals[a], vals[b])
            vals[a], vals[b] = hi, lo
    return vals


CHALF = C // 2
HBATCH = CHALF // K


def _sc_thresholds(x_hbm, thr_hbm, bufs, thr_buf, state, sems, out_sem):
    wid = lax.axis_index("s") * NC + lax.axis_index("c")

    def issue(q, h):
        pltpu.async_copy(
            x_hbm.at[wid, pl.ds(h * CHALF, CHALF), pl.ds(q * CHUNK, CHUNK)],
            bufs.at[h], sems.at[h])

    def wait(q, h):
        pltpu.make_async_copy(
            x_hbm.at[wid, pl.ds(h * CHALF, CHALF), pl.ds(q * CHUNK, CHUNK)],
            bufs.at[h], sems.at[h]).wait()

    issue(0, 0)

    @pl.loop(0, NCHUNKS)
    def _chunk(q):
        for h in range(2):
            wait(q, h)
            if h == 0:
                issue(q, 1)
            else:
                @pl.when(q + 1 < NCHUNKS)
                def _refill():
                    issue(q + 1, 0)

            buf = bufs.at[h]
            neg = jnp.full((LANES,), -jnp.inf, jnp.float32)

            for gbase in range(0, GROUPS, 2):
                def batch_body(c8, ms, gbase=gbase, buf=buf):
                    ms = list(ms)
                    base = c8 * K
                    for gg in range(2):
                        g = gbase + gg
                        t = [buf[base + k, g * LANES:(g + 1) * LANES]
                             for k in range(K)]
                        t = _apply_net(t, _SORT8)
                        m = ms[gg * K:(gg + 1) * K]
                        u = [jnp.maximum(m[i], t[K - 1 - i])
                             for i in range(K)]
                        u = _apply_net(u, _BITONIC8)
                        ms[gg * K:(gg + 1) * K] = u
                    return tuple(ms)

                if h == 0:
                    init = tuple(neg for _ in range(2 * K))
                else:
                    init = tuple(
                        state[(idx % K),
                              (gbase + idx // K) * LANES:
                              (gbase + idx // K) * LANES + LANES]
                        for idx in range(2 * K))

                ms = plsc.parallel_loop(
                    0, HBATCH, carry=init)(batch_body)

                if h == 0:
                    for idx in range(2 * K):
                        state[(idx % K),
                              (gbase + idx // K) * LANES:
                              (gbase + idx // K) * LANES + LANES] = ms[idx]
                else:
                    for gg in range(2):
                        g = gbase + gg
                        thr_buf[pl.ds(q * CHUNK + g * LANES, LANES)] = (
                            ms[gg * K + K - 1])

    pltpu.async_copy(thr_buf, thr_hbm.at[wid], out_sem).wait()


def _tc_mask(x_ref, thr_ref, out_ref):
    xb = x_ref[0]
    t = thr_ref[0]
    out_ref[0] = jnp.where(xb >= t, xb, jnp.float32(0.0))


@jax.jit
def kernel(x):
    xr = x.reshape(B, C, P)
    thr = pl.kernel(
        _sc_thresholds,
        out_type=jax.ShapeDtypeStruct((B, P), jnp.float32),
        mesh=plsc.VectorSubcoreMesh(core_axis_name="c", subcore_axis_name="s"),
        scratch_types=[
            pltpu.VMEM((2, CHALF, CHUNK), jnp.float32),
            pltpu.VMEM((P,), jnp.float32),
            pltpu.VMEM((K, GROUPS * LANES), jnp.float32),
            pltpu.SemaphoreType.DMA((2,)),
            pltpu.SemaphoreType.DMA,
        ],
        compiler_params=pltpu.CompilerParams(use_tc_tiling_on_sc=True),
    )(xr)
    out = pl.pallas_call(
        _tc_mask,
        grid=(B, C // CBLK),
        in_specs=[
            pl.BlockSpec((1, CBLK, P), lambda b, cb: (b, cb, 0)),
            pl.BlockSpec((1, 1, P), lambda b, cb: (b, 0, 0)),
        ],
        out_specs=pl.BlockSpec((1, CBLK, P), lambda b, cb: (b, cb, 0)),
        out_shape=jax.ShapeDtypeStruct((B, C, P), jnp.float32),
    )(xr, thr.reshape(B, 1, P))
    return out.reshape(B, C, H, W)


# speedup vs baseline: 2.5958x; 1.1102x over previous
import jax
import jax.numpy as jnp
from jax import lax
from jax.experimental import pallas as pl
from jax.experimental.pallas import tpu as pltpu
from jax.experimental.pallas import tpu_sc as plsc

B, C, H, W = 32, 768, 32, 32
P = H * W
K = 8
LANES = 16
CHUNK = 128
GROUPS = CHUNK // LANES
NCHUNKS = P // CHUNK
CBATCH = C // K
NC, NS = 2, 16
CBLK = 256

_SORT8 = [[(0, 1), (2, 3), (4, 5), (6, 7)],
          [(0, 2), (1, 3), (4, 6), (5, 7)],
          [(1, 2), (5, 6)],
          [(0, 4), (1, 5), (2, 6), (3, 7)],
          [(2, 4), (3, 5)],
          [(1, 2), (3, 4), (5, 6)]]
_BITONIC8 = [[(0, 4), (1, 5), (2, 6), (3, 7)],
             [(0, 2), (1, 3), (4, 6), (5, 7)],
             [(0, 1), (2, 3), (4, 5), (6, 7)]]


def _apply_net(vals, net):
    for layer in net:
        for a, b in layer:
            hi = jnp.maximum(vals[a], vals[b])
            lo = jnp.minimum(vals[a], vals[b])
            vals[a], vals[b] = hi, lo
    return vals


CHALF = C // 2
HBATCH = CHALF // K


def _sc_thresholds(x_hbm, thr_hbm, bufs, thr_buf, state, sems, out_sem):
    wid = lax.axis_index("s") * NC + lax.axis_index("c")

    def issue(q, h):
        pltpu.async_copy(
            x_hbm.at[wid, pl.ds(h * CHALF, CHALF), pl.ds(q * CHUNK, CHUNK)],
            bufs.at[h], sems.at[h])

    def wait(q, h):
        pltpu.make_async_copy(
            x_hbm.at[wid, pl.ds(h * CHALF, CHALF), pl.ds(q * CHUNK, CHUNK)],
            bufs.at[h], sems.at[h]).wait()

    issue(0, 0)

    @pl.loop(0, NCHUNKS)
    def _chunk(q):
        for h in range(2):
            wait(q, h)
            if h == 0:
                issue(q, 1)
            else:
                @pl.when(q + 1 < NCHUNKS)
                def _refill():
                    issue(q + 1, 0)

            buf = bufs.at[h]
            neg = jnp.full((LANES,), -jnp.inf, jnp.float32)

            for gbase in range(0, GROUPS, 2):
                def batch_body(c8, ms, gbase=gbase, buf=buf):
                    ms = list(ms)
                    base = c8 * K
                    for gg in range(2):
                        g = gbase + gg
                        t = [buf[base + k, g * LANES:(g + 1) * LANES]
                             for k in range(K)]
                        t = _apply_net(t, _SORT8)
                        m = ms[gg * K:(gg + 1) * K]
                        u = [jnp.maximum(m[i], t[K - 1 - i])
                             for i in range(K)]
                        u = _apply_net(u, _BITONIC8)
                        ms[gg * K:(gg + 1) * K] = u
                    return tuple(ms)

                if h == 0:
                    init = tuple(neg for _ in range(2 * K))
                else:
                    init = tuple(
                        state[(idx % K),
                              (gbase + idx // K) * LANES:
                              (gbase + idx // K) * LANES + LANES]
                        for idx in range(2 * K))

                ms = plsc.parallel_loop(
                    0, HBATCH, carry=init)(batch_body)

                if h == 0:
                    for idx in range(2 * K):
                        state[(idx % K),
                              (gbase + idx // K) * LANES:
                              (gbase + idx // K) * LANES + LANES] = ms[idx]
                else:
                    for gg in range(2):
                        g = gbase + gg
                        thr_buf[pl.ds(q * CHUNK + g * LANES, LANES)] = (
                            ms[gg * K + K - 1])

    pltpu.async_copy(thr_buf, thr_hbm.at[wid], out_sem).wait()


def _tc_mask(x_ref, thr_ref, out_ref):
    xb = x_ref[0]
    t = thr_ref[0]
    out_ref[0] = jnp.where(xb >= t, xb, jnp.float32(0.0))


@jax.jit
def kernel(x):
    xr = x.reshape(B, C, P)
    thr = pl.kernel(
        _sc_thresholds,
        out_type=jax.ShapeDtypeStruct((B, P), jnp.float32),
        mesh=plsc.VectorSubcoreMesh(core_axis_name="c", subcore_axis_name="s"),
        scratch_types=[
            pltpu.VMEM((2, CHALF, CHUNK), jnp.float32),
            pltpu.VMEM((P,), jnp.float32),
            pltpu.VMEM((K, GROUPS * LANES), jnp.float32),
            pltpu.SemaphoreType.DMA((2,)),
            pltpu.SemaphoreType.DMA,
        ],
        compiler_params=pltpu.CompilerParams(use_tc_tiling_on_sc=True),
    )(xr)
    out = pl.pallas_call(
        _tc_mask,
        grid=(B, C // CBLK),
        in_specs=[
            pl.BlockSpec((1, CBLK, P), lambda b, cb: (b, cb, 0)),
            pl.BlockSpec((1, 1, P), lambda b, cb: (b, 0, 0)),
        ],
        out_specs=pl.BlockSpec((1, CBLK, P), lambda b, cb: (b, cb, 0)),
        out_shape=jax.ShapeDtypeStruct((B, C, P), jnp.float32),
    )(xr, thr.reshape(B, 1, P))
    return out.reshape(B, C, H, W)


# speedup vs baseline: 2.7433x; 1.0568x over previous
import jax
import jax.numpy as jnp
from jax import lax
from jax.experimental import pallas as pl
from jax.experimental.pallas import tpu as pltpu
from jax.experimental.pallas import tpu_sc as plsc

B, C, H, W = 32, 768, 32, 32
P = H * W
K = 8
LANES = 16
CHUNK = 128
GROUPS = CHUNK // LANES
NCHUNKS = P // CHUNK
CBATCH = C // K
NC, NS = 2, 16
CBLK = 384

_SORT8 = [[(0, 1), (2, 3), (4, 5), (6, 7)],
          [(0, 2), (1, 3), (4, 6), (5, 7)],
          [(1, 2), (5, 6)],
          [(0, 4), (1, 5), (2, 6), (3, 7)],
          [(2, 4), (3, 5)],
          [(1, 2), (3, 4), (5, 6)]]
_BITONIC8 = [[(0, 4), (1, 5), (2, 6), (3, 7)],
             [(0, 2), (1, 3), (4, 6), (5, 7)],
             [(0, 1), (2, 3), (4, 5), (6, 7)]]


def _apply_net(vals, net):
    for layer in net:
        for a, b in layer:
            hi = jnp.maximum(vals[a], vals[b])
            lo = jnp.minimum(vals[a], vals[b])
            vals[a], vals[b] = hi, lo
    return vals


CHALF = C // 2
HBATCH = CHALF // K


def _sc_thresholds(x_hbm, thr_hbm, bufs, thr_buf, state, sems, out_sem):
    wid = lax.axis_index("s") * NC + lax.axis_index("c")

    def issue(q, h):
        pltpu.async_copy(
            x_hbm.at[wid, pl.ds(h * CHALF, CHALF), pl.ds(q * CHUNK, CHUNK)],
            bufs.at[h], sems.at[h])

    def wait(q, h):
        pltpu.make_async_copy(
            x_hbm.at[wid, pl.ds(h * CHALF, CHALF), pl.ds(q * CHUNK, CHUNK)],
            bufs.at[h], sems.at[h]).wait()

    issue(0, 0)

    @pl.loop(0, NCHUNKS)
    def _chunk(q):
        for h in range(2):
            wait(q, h)
            if h == 0:
                issue(q, 1)
            else:
                @pl.when(q + 1 < NCHUNKS)
                def _refill():
                    issue(q + 1, 0)

            buf = bufs.at[h]
            neg = jnp.full((LANES,), -jnp.inf, jnp.float32)

            for gbase in range(0, GROUPS, 2):
                def batch_body(c8, ms, gbase=gbase, buf=buf):
                    ms = list(ms)
                    base = c8 * K
                    for gg in range(2):
                        g = gbase + gg
                        t = [buf[base + k, g * LANES:(g + 1) * LANES]
                             for k in range(K)]
                        t = _apply_net(t, _SORT8)
                        m = ms[gg * K:(gg + 1) * K]
                        u = [jnp.maximum(m[i], t[K - 1 - i])
                             for i in range(K)]
                        u = _apply_net(u, _BITONIC8)
                        ms[gg * K:(gg + 1) * K] = u
                    return tuple(ms)

                if h == 0:
                    init = tuple(neg for _ in range(2 * K))
                else:
                    init = tuple(
                        state[(idx % K),
                              (gbase + idx // K) * LANES:
                              (gbase + idx // K) * LANES + LANES]
                        for idx in range(2 * K))

                ms = plsc.parallel_loop(
                    0, HBATCH, carry=init)(batch_body)

                if h == 0:
                    for idx in range(2 * K):
                        state[(idx % K),
                              (gbase + idx // K) * LANES:
                              (gbase + idx // K) * LANES + LANES] = ms[idx]
                else:
                    for gg in range(2):
                        g = gbase + gg
                        thr_buf[pl.ds(q * CHUNK + g * LANES, LANES)] = (
                            ms[gg * K + K - 1])

    pltpu.async_copy(thr_buf, thr_hbm.at[wid], out_sem).wait()


def _tc_mask(x_ref, thr_ref, out_ref):
    xb = x_ref[0]
    t = thr_ref[0]
    out_ref[0] = jnp.where(xb >= t, xb, jnp.float32(0.0))


@jax.jit
def kernel(x):
    xr = x.reshape(B, C, P)
    thr = pl.kernel(
        _sc_thresholds,
        out_type=jax.ShapeDtypeStruct((B, P), jnp.float32),
        mesh=plsc.VectorSubcoreMesh(core_axis_name="c", subcore_axis_name="s"),
        scratch_types=[
            pltpu.VMEM((2, CHALF, CHUNK), jnp.float32),
            pltpu.VMEM((P,), jnp.float32),
            pltpu.VMEM((K, GROUPS * LANES), jnp.float32),
            pltpu.SemaphoreType.DMA((2,)),
            pltpu.SemaphoreType.DMA,
        ],
        compiler_params=pltpu.CompilerParams(use_tc_tiling_on_sc=True),
    )(xr)
    out = pl.pallas_call(
        _tc_mask,
        grid=(B, C // CBLK),
        in_specs=[
            pl.BlockSpec((1, CBLK, P), lambda b, cb: (b, cb, 0)),
            pl.BlockSpec((1, 1, P), lambda b, cb: (b, 0, 0)),
        ],
        out_specs=pl.BlockSpec((1, CBLK, P), lambda b, cb: (b, cb, 0)),
        out_shape=jax.ShapeDtypeStruct((B, C, P), jnp.float32),
    )(xr, thr.reshape(B, 1, P))
    return out.reshape(B, C, H, W)
